# Initial kernel scaffold; baseline (speedup 1.0000x reference)
import jax
import jax.numpy as jnp
from jax.experimental import pallas as pl


def kernel(champions, roles, champ_table, role_table, W1, b1, W2, b2):
    raise NotImplementedError("write your pallas kernel here")


# speedup vs baseline: 7.7015x; 7.7015x over previous
import functools

import jax
import jax.numpy as jnp
import numpy as np
from jax import lax
from jax.experimental import pallas as pl
from jax.experimental.pallas import tpu as pltpu
from jax.experimental.pallas import tpu_sc as plsc

NUM_CHAMPS = 164
NUM_ROLES = 6
PAIRS = NUM_CHAMPS * NUM_ROLES
OUT = 10
LANES = 16
NC, NS = 2, 16
NW = NC * NS


def _pair_table_body(ct, rt, w1a, w1b, b1, w2, b2, out):
    cp = jnp.dot(ct[...], w1a[...], preferred_element_type=jnp.float32)
    rp = jnp.dot(rt[...], w1b[...], preferred_element_type=jnp.float32)
    pc = lax.broadcasted_iota(jnp.int32, (PAIRS, NUM_CHAMPS), 0) // NUM_ROLES
    ec = (pc == lax.broadcasted_iota(jnp.int32, (PAIRS, NUM_CHAMPS), 1)).astype(jnp.float32)
    pr = lax.broadcasted_iota(jnp.int32, (PAIRS, 8), 0) % NUM_ROLES
    er = (pr == lax.broadcasted_iota(jnp.int32, (PAIRS, 8), 1)).astype(jnp.float32)
    h = jnp.maximum(
        jnp.dot(ec, cp, preferred_element_type=jnp.float32)
        + jnp.dot(er, rp, preferred_element_type=jnp.float32)
        + b1[...],
        0.0,
    )
    out[...] = jnp.dot(h, w2[...], preferred_element_type=jnp.float32) + b2[...]


def _build_pair_table(champ_table, role_table, W1, b1, W2, b2):
    rt_pad = jnp.pad(role_table, ((0, 8 - NUM_ROLES), (0, 0)))
    return pl.pallas_call(
        _pair_table_body,
        out_shape=jax.ShapeDtypeStruct((PAIRS, OUT), jnp.float32),
    )(champ_table, rt_pad, W1[:32], W1[32:], b1.reshape(1, -1), W2,
      b2.reshape(1, -1))


def _gather_body(per_w, chunk, table_hbm, champ_hbm, role_hbm, out_hbm,
                 table_v, cbuf, rbuf, tbuf, obuf):
    wid = lax.axis_index("s") * NC + lax.axis_index("c")
    base = wid * per_w
    pltpu.sync_copy(table_hbm, table_v)
    def splat(x):
        return jnp.full((16,), x, jnp.int32)

    lane = lax.iota(jnp.int32, 16)
    toffs, cols = [], []
    for k in range(10):
        p = lane + splat(16 * k)
        t = p // splat(10)
        toffs.append(t)
        cols.append(p - t * splat(10))

    def chunk_body(i, _):
        tb = base + i * chunk
        pltpu.sync_copy(champ_hbm.at[pl.ds(tb, chunk)], cbuf)
        pltpu.sync_copy(role_hbm.at[pl.ds(tb, chunk)], rbuf)

        def p1(g, _):
            c = cbuf[pl.ds(g * 16, 16)]
            r = rbuf[pl.ds(g * 16, 16)]
            tbuf[pl.ds(g * 16, 16)] = (c * splat(NUM_ROLES) + r) * splat(OUT)
            return 0

        lax.fori_loop(0, chunk // 16, p1, 0)

        def p2(g, _):
            gbase = splat(g * 16)
            for k in range(10):
                t = plsc.load_gather(tbuf, [gbase + toffs[k]])
                v = plsc.load_gather(table_v, [t + cols[k]])
                obuf[pl.ds(g * 160 + k * 16, 16)] = v
            return 0

        lax.fori_loop(0, chunk // 16, p2, 0)
        pltpu.sync_copy(obuf, out_hbm.at[pl.ds(tb * 10, chunk * 10)])
        return 0

    lax.fori_loop(0, per_w // chunk, chunk_body, 0)


@functools.cache
def _make_gather(n_tokens):
    assert n_tokens % NW == 0
    per_w = n_tokens // NW
    chunk = 4096
    assert per_w % chunk == 0
    mesh = plsc.VectorSubcoreMesh(core_axis_name="c", subcore_axis_name="s")
    return pl.kernel(
        functools.partial(_gather_body, per_w, chunk),
        out_type=jax.ShapeDtypeStruct((n_tokens * OUT,), jnp.float32),
        mesh=mesh,
        compiler_params=pltpu.CompilerParams(needs_layout_passes=False),
        scratch_types=[
            pltpu.VMEM((PAIRS * OUT,), jnp.float32),
            pltpu.VMEM((chunk,), jnp.int32),
            pltpu.VMEM((chunk,), jnp.int32),
            pltpu.VMEM((chunk,), jnp.int32),
            pltpu.VMEM((chunk * 10,), jnp.float32),
        ],
    )


def kernel(champions, roles, champ_table, role_table, W1, b1, W2, b2):
    B, L = champions.shape
    pair_table = _build_pair_table(champ_table, role_table, W1, b1, W2, b2)
    gather = _make_gather(B * L)
    out_flat = gather(pair_table.reshape(-1), champions.reshape(-1),
                      roles.reshape(-1))
    return out_flat.reshape(B, L, OUT)


# speedup vs baseline: 7.7331x; 1.0041x over previous
import functools

import jax
import jax.numpy as jnp
import numpy as np
from jax import lax
from jax.experimental import pallas as pl
from jax.experimental.pallas import tpu as pltpu
from jax.experimental.pallas import tpu_sc as plsc

NUM_CHAMPS = 164
NUM_ROLES = 6
PAIRS = NUM_CHAMPS * NUM_ROLES
OUT = 10
LANES = 16
NC, NS = 2, 16
NW = NC * NS


def _pair_table_body(ct, rt, w1a, w1b, b1, w2, b2, out):
    cp = jnp.dot(ct[...], w1a[...], preferred_element_type=jnp.float32)
    rp = jnp.dot(rt[...], w1b[...], preferred_element_type=jnp.float32)
    pc = lax.broadcasted_iota(jnp.int32, (PAIRS, NUM_CHAMPS), 0) // NUM_ROLES
    ec = (pc == lax.broadcasted_iota(jnp.int32, (PAIRS, NUM_CHAMPS), 1)).astype(jnp.float32)
    pr = lax.broadcasted_iota(jnp.int32, (PAIRS, 8), 0) % NUM_ROLES
    er = (pr == lax.broadcasted_iota(jnp.int32, (PAIRS, 8), 1)).astype(jnp.float32)
    h = jnp.maximum(
        jnp.dot(ec, cp, preferred_element_type=jnp.float32)
        + jnp.dot(er, rp, preferred_element_type=jnp.float32)
        + b1[...],
        0.0,
    )
    out[...] = jnp.dot(h, w2[...], preferred_element_type=jnp.float32) + b2[...]


def _build_pair_table(champ_table, role_table, W1, b1, W2, b2):
    rt_pad = jnp.pad(role_table, ((0, 8 - NUM_ROLES), (0, 0)))
    return pl.pallas_call(
        _pair_table_body,
        out_shape=jax.ShapeDtypeStruct((PAIRS, OUT), jnp.float32),
    )(champ_table, rt_pad, W1[:32], W1[32:], b1.reshape(1, -1), W2,
      b2.reshape(1, -1))


def _gather_body(rows_per_w, row_chunk, seq_len, table_hbm, champ_hbm,
                 role_hbm, out_hbm, table_v, cbuf, rbuf, tbuf, obuf,
                 rowtab, coltab):
    chunk = row_chunk * seq_len
    wid = lax.axis_index("s") * NC + lax.axis_index("c")
    row_base = wid * rows_per_w
    pltpu.sync_copy(table_hbm, table_v)
    def splat(x):
        return jnp.full((16,), x, jnp.int32)

    lane = lax.iota(jnp.int32, 16)
    toffs, cols = [], []
    for k in range(10):
        p = lane + splat(16 * k)
        t = p // splat(10)
        toffs.append(t)
        cols.append(p - t * splat(10))

    def fill(g, _):
        tok = splat(g * 16) + lane
        r = tok // splat(seq_len)
        rowtab[pl.ds(g * 16, 16)] = r
        coltab[pl.ds(g * 16, 16)] = tok - r * splat(seq_len)
        return 0

    lax.fori_loop(0, chunk // 16, fill, 0)

    def chunk_body(i, _):
        r0 = row_base + i * row_chunk
        tb = r0 * seq_len
        pltpu.sync_copy(champ_hbm.at[pl.ds(r0, row_chunk), :], cbuf)
        pltpu.sync_copy(role_hbm.at[pl.ds(r0, row_chunk), :], rbuf)

        def p1(g, _):
            row = rowtab[pl.ds(g * 16, 16)]
            col = coltab[pl.ds(g * 16, 16)]
            c = plsc.load_gather(cbuf, [row, col])
            r = plsc.load_gather(rbuf, [row, col])
            tbuf[pl.ds(g * 16, 16)] = (c * splat(NUM_ROLES) + r) * splat(OUT)
            return 0

        lax.fori_loop(0, chunk // 16, p1, 0)

        def p2(g, _):
            gbase = splat(g * 16)
            for k in range(10):
                t = plsc.load_gather(tbuf, [gbase + toffs[k]])
                v = plsc.load_gather(table_v, [t + cols[k]])
                obuf[pl.ds(g * 160 + k * 16, 16)] = v
            return 0

        lax.fori_loop(0, chunk // 16, p2, 0)
        pltpu.sync_copy(obuf, out_hbm.at[pl.ds(tb * 10, chunk * 10)])
        return 0

    lax.fori_loop(0, rows_per_w // row_chunk, chunk_body, 0)


@functools.cache
def _make_gather(n_rows, seq_len):
    assert n_rows % NW == 0
    rows_per_w = n_rows // NW
    row_chunk = 32
    assert rows_per_w % row_chunk == 0
    chunk = row_chunk * seq_len
    assert chunk % 16 == 0
    mesh = plsc.VectorSubcoreMesh(core_axis_name="c", subcore_axis_name="s")
    return pl.kernel(
        functools.partial(_gather_body, rows_per_w, row_chunk, seq_len),
        out_type=jax.ShapeDtypeStruct((n_rows * seq_len * OUT,), jnp.float32),
        mesh=mesh,
        compiler_params=pltpu.CompilerParams(needs_layout_passes=False),
        scratch_types=[
            pltpu.VMEM((PAIRS * OUT,), jnp.float32),
            pltpu.VMEM((row_chunk, seq_len), jnp.int32),
            pltpu.VMEM((row_chunk, seq_len), jnp.int32),
            pltpu.VMEM((chunk,), jnp.int32),
            pltpu.VMEM((chunk * 10,), jnp.float32),
            pltpu.VMEM((chunk,), jnp.int32),
            pltpu.VMEM((chunk,), jnp.int32),
        ],
    )


def kernel(champions, roles, champ_table, role_table, W1, b1, W2, b2):
    B, L = champions.shape
    pair_table = _build_pair_table(champ_table, role_table, W1, b1, W2, b2)
    gather = _make_gather(B, L)
    out_flat = gather(pair_table.reshape(-1), champions, roles)
    return out_flat.reshape(B, L, OUT)


# speedup vs baseline: 56.5995x; 7.3191x over previous
import functools

import jax
import jax.numpy as jnp
from jax import lax
from jax.experimental import pallas as pl
from jax.experimental.pallas import tpu as pltpu
from jax.experimental.pallas import tpu_sc as plsc

NUM_CHAMPS = 164
NUM_ROLES = 6
PAIRS = NUM_CHAMPS * NUM_ROLES
OUT = 10
NC, NS = 2, 16
NW = NC * NS

ROWS = 8
COLS = 512


def _pair_table_body(ct, rt, w1a, w1b, b1, w2, b2, out):
    cp = jnp.dot(ct[...], w1a[...], preferred_element_type=jnp.float32)
    rp = jnp.dot(rt[...], w1b[...], preferred_element_type=jnp.float32)
    pc = lax.broadcasted_iota(jnp.int32, (PAIRS, NUM_CHAMPS), 0) // NUM_ROLES
    ec = (pc == lax.broadcasted_iota(jnp.int32, (PAIRS, NUM_CHAMPS), 1)).astype(jnp.float32)
    pr = lax.broadcasted_iota(jnp.int32, (PAIRS, 8), 0) % NUM_ROLES
    er = (pr == lax.broadcasted_iota(jnp.int32, (PAIRS, 8), 1)).astype(jnp.float32)
    h = jnp.maximum(
        jnp.dot(ec, cp, preferred_element_type=jnp.float32)
        + jnp.dot(er, rp, preferred_element_type=jnp.float32)
        + b1[...],
        0.0,
    )
    out[...] = jnp.dot(h, w2[...], preferred_element_type=jnp.float32) + b2[...]


def _build_pair_table(champ_table, role_table, W1, b1, W2, b2):
    rt_pad = jnp.pad(role_table, ((0, 8 - NUM_ROLES), (0, 0)))
    return pl.pallas_call(
        _pair_table_body,
        out_shape=jax.ShapeDtypeStruct((PAIRS, OUT), jnp.float32),
    )(champ_table, rt_pad, W1[:32], W1[32:], b1.reshape(1, -1), W2,
      b2.reshape(1, -1))


def _gather_body(chunks_per_w, col_blocks, table_hbm, champ_hbm, role_hbm,
                 out_hbm, table_v, cbuf, rbuf, obuf):
    wid = lax.axis_index("s") * NC + lax.axis_index("c")
    pltpu.sync_copy(table_hbm, table_v)

    def splat(x):
        return jnp.full((16,), x, jnp.int32)

    def chunk_body(i, _):
        t = wid * chunks_per_w + i
        lb = t // col_blocks
        bb = t - lb * col_blocks
        l0 = lb * ROWS
        b0 = bb * COLS
        pltpu.sync_copy(champ_hbm.at[pl.ds(l0, ROWS), pl.ds(b0, COLS)], cbuf)
        pltpu.sync_copy(role_hbm.at[pl.ds(l0, ROWS), pl.ds(b0, COLS)], rbuf)

        def row_body(r, _):
            def grp_body(g, _):
                c = cbuf[r, pl.ds(g * 16, 16)]
                rr = rbuf[r, pl.ds(g * 16, 16)]
                idx = (c * splat(NUM_ROLES) + rr) * splat(OUT)
                for d in range(OUT):
                    v = plsc.load_gather(table_v, [idx + splat(d)])
                    obuf[d, r, pl.ds(g * 16, 16)] = v
                return 0

            lax.fori_loop(0, COLS // 16, grp_body, 0)
            return 0

        lax.fori_loop(0, ROWS, row_body, 0)
        pltpu.sync_copy(
            obuf, out_hbm.at[:, pl.ds(l0, ROWS), pl.ds(b0, COLS)])
        return 0

    lax.fori_loop(0, chunks_per_w, chunk_body, 0)


@functools.cache
def _make_gather(batch, seq_len):
    n_chunks = (seq_len // ROWS) * (batch // COLS)
    assert n_chunks % NW == 0
    chunks_per_w = n_chunks // NW
    col_blocks = batch // COLS
    mesh = plsc.VectorSubcoreMesh(core_axis_name="c", subcore_axis_name="s")
    return pl.kernel(
        functools.partial(_gather_body, chunks_per_w, col_blocks),
        out_type=jax.ShapeDtypeStruct((OUT, seq_len, batch), jnp.float32),
        mesh=mesh,
        compiler_params=pltpu.CompilerParams(needs_layout_passes=False),
        scratch_types=[
            pltpu.VMEM((PAIRS * OUT,), jnp.float32),
            pltpu.VMEM((ROWS, COLS), jnp.int32),
            pltpu.VMEM((ROWS, COLS), jnp.int32),
            pltpu.VMEM((OUT, ROWS, COLS), jnp.float32),
        ],
    )


def kernel(champions, roles, champ_table, role_table, W1, b1, W2, b2):
    B, L = champions.shape
    pair_table = _build_pair_table(champ_table, role_table, W1, b1, W2, b2)
    gather = _make_gather(B, L)
    out_t = gather(pair_table.reshape(-1), jnp.swapaxes(champions, 0, 1),
                   jnp.swapaxes(roles, 0, 1))
    return jnp.transpose(out_t, (2, 1, 0))


# speedup vs baseline: 136.3588x; 2.4092x over previous
import functools

import jax
import jax.numpy as jnp
from jax import lax
from jax.experimental import pallas as pl
from jax.experimental.pallas import tpu as pltpu
from jax.experimental.pallas import tpu_sc as plsc

NUM_CHAMPS = 164
NUM_ROLES = 6
PAIRS = NUM_CHAMPS * NUM_ROLES
OUT = 10
NC, NS = 2, 16
NW = NC * NS

ROWS = 8
COLS = 512


def _pair_table_body(ct, rt, w1a, w1b, b1, w2, b2, out):
    cp = jnp.dot(ct[...], w1a[...], preferred_element_type=jnp.float32)
    rp = jnp.dot(rt[...], w1b[...], preferred_element_type=jnp.float32)
    pc = lax.broadcasted_iota(jnp.int32, (PAIRS, NUM_CHAMPS), 0) // NUM_ROLES
    ec = (pc == lax.broadcasted_iota(jnp.int32, (PAIRS, NUM_CHAMPS), 1)).astype(jnp.float32)
    pr = lax.broadcasted_iota(jnp.int32, (PAIRS, 8), 0) % NUM_ROLES
    er = (pr == lax.broadcasted_iota(jnp.int32, (PAIRS, 8), 1)).astype(jnp.float32)
    h = jnp.maximum(
        jnp.dot(ec, cp, preferred_element_type=jnp.float32)
        + jnp.dot(er, rp, preferred_element_type=jnp.float32)
        + b1[...],
        0.0,
    )
    out[...] = jnp.dot(h, w2[...], preferred_element_type=jnp.float32) + b2[...]


def _build_pair_table(champ_table, role_table, W1, b1, W2, b2):
    rt_pad = jnp.pad(role_table, ((0, 8 - NUM_ROLES), (0, 0)))
    return pl.pallas_call(
        _pair_table_body,
        out_shape=jax.ShapeDtypeStruct((PAIRS, OUT), jnp.float32),
    )(champ_table, rt_pad, W1[:32], W1[32:], b1.reshape(1, -1), W2,
      b2.reshape(1, -1))


def _gather_body(chunks_per_w, col_blocks, table_hbm, champ_hbm, role_hbm,
                 out_hbm, table_v, cbuf, rbuf, obuf):
    wid = lax.axis_index("s") * NC + lax.axis_index("c")
    pltpu.sync_copy(table_hbm, table_v)

    def splat(x):
        return jnp.full((16,), x, jnp.int32)

    def chunk_body(i, _):
        t = wid * chunks_per_w + i
        lb = t // col_blocks
        bb = t - lb * col_blocks
        l0 = lb * ROWS
        b0 = bb * COLS
        pltpu.sync_copy(champ_hbm.at[pl.ds(l0, ROWS), pl.ds(b0, COLS)], cbuf)
        pltpu.sync_copy(role_hbm.at[pl.ds(l0, ROWS), pl.ds(b0, COLS)], rbuf)

        def row_body(r, _):
            @plsc.parallel_loop(0, COLS // 16, unroll=4)
            def grp_body(g):
                c = cbuf[r, pl.ds(g * 16, 16)]
                rr = rbuf[r, pl.ds(g * 16, 16)]
                idx = (c * splat(NUM_ROLES) + rr) * splat(OUT)
                for d in range(OUT):
                    v = plsc.load_gather(table_v, [idx + splat(d)])
                    obuf[d, r, pl.ds(g * 16, 16)] = v

            return 0

        lax.fori_loop(0, ROWS, row_body, 0)
        pltpu.sync_copy(
            obuf, out_hbm.at[:, pl.ds(l0, ROWS), pl.ds(b0, COLS)])
        return 0

    lax.fori_loop(0, chunks_per_w, chunk_body, 0)


@functools.cache
def _make_gather(batch, seq_len):
    n_chunks = (seq_len // ROWS) * (batch // COLS)
    assert n_chunks % NW == 0
    chunks_per_w = n_chunks // NW
    col_blocks = batch // COLS
    mesh = plsc.VectorSubcoreMesh(core_axis_name="c", subcore_axis_name="s")
    return pl.kernel(
        functools.partial(_gather_body, chunks_per_w, col_blocks),
        out_type=jax.ShapeDtypeStruct((OUT, seq_len, batch), jnp.float32),
        mesh=mesh,
        compiler_params=pltpu.CompilerParams(needs_layout_passes=False),
        scratch_types=[
            pltpu.VMEM((PAIRS * OUT,), jnp.float32),
            pltpu.VMEM((ROWS, COLS), jnp.int32),
            pltpu.VMEM((ROWS, COLS), jnp.int32),
            pltpu.VMEM((OUT, ROWS, COLS), jnp.float32),
        ],
    )


def kernel(champions, roles, champ_table, role_table, W1, b1, W2, b2):
    B, L = champions.shape
    pair_table = _build_pair_table(champ_table, role_table, W1, b1, W2, b2)
    gather = _make_gather(B, L)
    out_t = gather(pair_table.reshape(-1), jnp.swapaxes(champions, 0, 1),
                   jnp.swapaxes(roles, 0, 1))
    return jnp.transpose(out_t, (2, 1, 0))


# speedup vs baseline: 232.2717x; 1.7034x over previous
import functools

import jax
import jax.numpy as jnp
from jax import lax
from jax.experimental import pallas as pl
from jax.experimental.pallas import tpu as pltpu
from jax.experimental.pallas import tpu_sc as plsc

NUM_CHAMPS = 164
NUM_ROLES = 6
PAIRS = NUM_CHAMPS * NUM_ROLES
OUT = 10
NC, NS = 2, 16
NW = NC * NS

ROWS = 8
COLS = 512


def _pair_table_body(ct, rt, w1a, w1b, b1, w2, b2, out):
    cp = jnp.dot(ct[...], w1a[...], preferred_element_type=jnp.float32)
    rp = jnp.dot(rt[...], w1b[...], preferred_element_type=jnp.float32)
    pc = lax.broadcasted_iota(jnp.int32, (PAIRS, NUM_CHAMPS), 0) // NUM_ROLES
    ec = (pc == lax.broadcasted_iota(jnp.int32, (PAIRS, NUM_CHAMPS), 1)).astype(jnp.float32)
    pr = lax.broadcasted_iota(jnp.int32, (PAIRS, 8), 0) % NUM_ROLES
    er = (pr == lax.broadcasted_iota(jnp.int32, (PAIRS, 8), 1)).astype(jnp.float32)
    h = jnp.maximum(
        jnp.dot(ec, cp, preferred_element_type=jnp.float32)
        + jnp.dot(er, rp, preferred_element_type=jnp.float32)
        + b1[...],
        0.0,
    )
    out[...] = jnp.dot(h, w2[...], preferred_element_type=jnp.float32) + b2[...]


def _build_pair_table(champ_table, role_table, W1, b1, W2, b2):
    rt_pad = jnp.pad(role_table, ((0, 8 - NUM_ROLES), (0, 0)))
    return pl.pallas_call(
        _pair_table_body,
        out_shape=jax.ShapeDtypeStruct((PAIRS, OUT), jnp.float32),
    )(champ_table, rt_pad, W1[:32], W1[32:], b1.reshape(1, -1), W2,
      b2.reshape(1, -1))


def _gather_body(chunks_per_w, col_blocks, table_hbm, champ_hbm, role_hbm,
                 out_hbm, table_v, cbuf, rbuf, obuf, csem, rsem, osem):
    wid = lax.axis_index("s") * NC + lax.axis_index("c")
    pltpu.sync_copy(table_hbm, table_v)

    def splat(x):
        return jnp.full((16,), x, jnp.int32)

    def offsets(i):
        t = wid * chunks_per_w + i
        lb = t // col_blocks
        bb = t - lb * col_blocks
        return lb * ROWS, bb * COLS

    def in_copies(i, slot):
        l0, b0 = offsets(i)
        src = lambda ref: ref.at[pl.ds(l0, ROWS), pl.ds(b0, COLS)]
        return (
            pltpu.make_async_copy(src(champ_hbm), cbuf.at[slot], csem.at[slot]),
            pltpu.make_async_copy(src(role_hbm), rbuf.at[slot], rsem.at[slot]),
        )

    def out_copy(i, slot):
        l0, b0 = offsets(i)
        return pltpu.make_async_copy(
            obuf.at[slot], out_hbm.at[:, pl.ds(l0, ROWS), pl.ds(b0, COLS)],
            osem.at[slot])

    for cp in in_copies(0, 0):
        cp.start()

    def chunk_body(i, _):
        slot = lax.rem(i, 2)
        nxt = 1 - slot

        @pl.when(i + 1 < chunks_per_w)
        def _():
            for cp in in_copies(i + 1, nxt):
                cp.start()

        for cp in in_copies(i, slot):
            cp.wait()

        @pl.when(i >= 2)
        def _():
            out_copy(i - 2, slot).wait()

        def row_body(r, _):
            @plsc.parallel_loop(0, COLS // 16, unroll=4)
            def grp_body(g):
                c = cbuf[slot, r, pl.ds(g * 16, 16)]
                rr = rbuf[slot, r, pl.ds(g * 16, 16)]
                idx = (c * splat(NUM_ROLES) + rr) * splat(OUT)
                for d in range(OUT):
                    v = plsc.load_gather(table_v, [idx + splat(d)])
                    obuf[slot, d, r, pl.ds(g * 16, 16)] = v

            return 0

        lax.fori_loop(0, ROWS, row_body, 0)
        out_copy(i, slot).start()
        return 0

    lax.fori_loop(0, chunks_per_w, chunk_body, 0)
    out_copy(chunks_per_w - 2, lax.rem(chunks_per_w - 2, 2)).wait()
    out_copy(chunks_per_w - 1, lax.rem(chunks_per_w - 1, 2)).wait()


@functools.cache
def _make_gather(batch, seq_len):
    n_chunks = (seq_len // ROWS) * (batch // COLS)
    assert n_chunks % NW == 0
    chunks_per_w = n_chunks // NW
    col_blocks = batch // COLS
    mesh = plsc.VectorSubcoreMesh(core_axis_name="c", subcore_axis_name="s")
    return pl.kernel(
        functools.partial(_gather_body, chunks_per_w, col_blocks),
        out_type=jax.ShapeDtypeStruct((OUT, seq_len, batch), jnp.float32),
        mesh=mesh,
        compiler_params=pltpu.CompilerParams(needs_layout_passes=False),
        scratch_types=[
            pltpu.VMEM((PAIRS * OUT,), jnp.float32),
            pltpu.VMEM((2, ROWS, COLS), jnp.int32),
            pltpu.VMEM((2, ROWS, COLS), jnp.int32),
            pltpu.VMEM((2, OUT, ROWS, COLS), jnp.float32),
            pltpu.SemaphoreType.DMA((2,)),
            pltpu.SemaphoreType.DMA((2,)),
            pltpu.SemaphoreType.DMA((2,)),
        ],
    )


def kernel(champions, roles, champ_table, role_table, W1, b1, W2, b2):
    B, L = champions.shape
    pair_table = _build_pair_table(champ_table, role_table, W1, b1, W2, b2)
    gather = _make_gather(B, L)
    out_t = gather(pair_table.reshape(-1), jnp.swapaxes(champions, 0, 1),
                   jnp.swapaxes(roles, 0, 1))
    return jnp.transpose(out_t, (2, 1, 0))


# speedup vs baseline: 260.9704x; 1.1236x over previous
import functools

import jax
import jax.numpy as jnp
from jax import lax
from jax.experimental import pallas as pl
from jax.experimental.pallas import tpu as pltpu
from jax.experimental.pallas import tpu_sc as plsc

NUM_CHAMPS = 164
NUM_ROLES = 6
CHAMP_DIM = 32
PAIRS = NUM_CHAMPS * NUM_ROLES
OUT = 10
NC, NS = 2, 16
NW = NC * NS

ROWS = 8
COLS = 512


def _pair_table_body(ctT, rtT, w1aT, w1bT, b1c, w2T, b2c, out):
    cpT = jnp.dot(w1aT[...], ctT[...], preferred_element_type=jnp.float32)
    rpT = jnp.dot(w1bT[...], rtT[...], preferred_element_type=jnp.float32)
    pc = lax.broadcasted_iota(jnp.int32, (NUM_CHAMPS, PAIRS), 1) // NUM_ROLES
    ec = (pc == lax.broadcasted_iota(jnp.int32, (NUM_CHAMPS, PAIRS), 0)).astype(jnp.float32)
    pr = lax.broadcasted_iota(jnp.int32, (8, PAIRS), 1) % NUM_ROLES
    er = (pr == lax.broadcasted_iota(jnp.int32, (8, PAIRS), 0)).astype(jnp.float32)
    hT = jnp.maximum(
        jnp.dot(cpT, ec, preferred_element_type=jnp.float32)
        + jnp.dot(rpT, er, preferred_element_type=jnp.float32)
        + b1c[...],
        0.0,
    )
    out[...] = jnp.dot(w2T[...], hT, preferred_element_type=jnp.float32) + b2c[...]


def _build_pair_table(champ_table, role_table, W1, b1, W2, b2):
    ctT = jnp.swapaxes(champ_table, 0, 1)
    rtT = jnp.pad(jnp.swapaxes(role_table, 0, 1), ((0, 0), (0, 8 - NUM_ROLES)))
    W1T = jnp.swapaxes(W1, 0, 1)
    w2T = jnp.swapaxes(W2, 0, 1)
    return pl.pallas_call(
        _pair_table_body,
        out_shape=jax.ShapeDtypeStruct((OUT, PAIRS), jnp.float32),
    )(ctT, rtT, W1T[:, :CHAMP_DIM], W1T[:, CHAMP_DIM:], b1.reshape(-1, 1),
      w2T, b2.reshape(-1, 1))


def _gather_body(chunks_per_w, col_blocks, table_hbm, champ_hbm, role_hbm,
                 out_hbm, table_v, cbuf, rbuf, obuf, csem, rsem, osem):
    wid = lax.axis_index("s") * NC + lax.axis_index("c")
    pltpu.sync_copy(table_hbm, table_v)

    def splat(x):
        return jnp.full((16,), x, jnp.int32)

    def offsets(i):
        t = wid * chunks_per_w + i
        lb = t // col_blocks
        bb = t - lb * col_blocks
        return lb * ROWS, bb * COLS

    def in_copies(i, slot):
        l0, b0 = offsets(i)
        src = lambda ref: ref.at[pl.ds(l0, ROWS), pl.ds(b0, COLS)]
        return (
            pltpu.make_async_copy(src(champ_hbm), cbuf.at[slot], csem.at[slot]),
            pltpu.make_async_copy(src(role_hbm), rbuf.at[slot], rsem.at[slot]),
        )

    def out_copy(i, slot):
        l0, b0 = offsets(i)
        return pltpu.make_async_copy(
            obuf.at[slot], out_hbm.at[:, pl.ds(l0, ROWS), pl.ds(b0, COLS)],
            osem.at[slot])

    for cp in in_copies(0, 0):
        cp.start()

    def chunk_body(i, _):
        slot = lax.rem(i, 2)
        nxt = 1 - slot

        @pl.when(i + 1 < chunks_per_w)
        def _():
            for cp in in_copies(i + 1, nxt):
                cp.start()

        for cp in in_copies(i, slot):
            cp.wait()

        @pl.when(i >= 2)
        def _():
            out_copy(i - 2, slot).wait()

        def row_body(r, _):
            @plsc.parallel_loop(0, COLS // 16, unroll=4)
            def grp_body(g):
                c = cbuf[slot, r, pl.ds(g * 16, 16)]
                rr = rbuf[slot, r, pl.ds(g * 16, 16)]
                idx = c * splat(NUM_ROLES) + rr
                for d in range(OUT):
                    v = plsc.load_gather(table_v, [idx + splat(d * PAIRS)])
                    obuf[slot, d, r, pl.ds(g * 16, 16)] = v

            return 0

        lax.fori_loop(0, ROWS, row_body, 0)
        out_copy(i, slot).start()
        return 0

    lax.fori_loop(0, chunks_per_w, chunk_body, 0)
    out_copy(chunks_per_w - 2, lax.rem(chunks_per_w - 2, 2)).wait()
    out_copy(chunks_per_w - 1, lax.rem(chunks_per_w - 1, 2)).wait()


@functools.cache
def _make_gather(batch, seq_len):
    n_chunks = (seq_len // ROWS) * (batch // COLS)
    assert n_chunks % NW == 0
    chunks_per_w = n_chunks // NW
    col_blocks = batch // COLS
    mesh = plsc.VectorSubcoreMesh(core_axis_name="c", subcore_axis_name="s")
    return pl.kernel(
        functools.partial(_gather_body, chunks_per_w, col_blocks),
        out_type=jax.ShapeDtypeStruct((OUT, seq_len, batch), jnp.float32),
        mesh=mesh,
        compiler_params=pltpu.CompilerParams(needs_layout_passes=False),
        scratch_types=[
            pltpu.VMEM((PAIRS * OUT,), jnp.float32),
            pltpu.VMEM((2, ROWS, COLS), jnp.int32),
            pltpu.VMEM((2, ROWS, COLS), jnp.int32),
            pltpu.VMEM((2, OUT, ROWS, COLS), jnp.float32),
            pltpu.SemaphoreType.DMA((2,)),
            pltpu.SemaphoreType.DMA((2,)),
            pltpu.SemaphoreType.DMA((2,)),
        ],
    )


def kernel(champions, roles, champ_table, role_table, W1, b1, W2, b2):
    B, L = champions.shape
    pair_table = _build_pair_table(champ_table, role_table, W1, b1, W2, b2)
    gather = _make_gather(B, L)
    out_t = gather(pair_table.reshape(-1), jnp.swapaxes(champions, 0, 1),
                   jnp.swapaxes(roles, 0, 1))
    return jnp.transpose(out_t, (2, 1, 0))


# speedup vs baseline: 272.3444x; 1.0436x over previous
import functools

import jax
import jax.numpy as jnp
from jax import lax
from jax.experimental import pallas as pl
from jax.experimental.pallas import tpu as pltpu
from jax.experimental.pallas import tpu_sc as plsc

NUM_CHAMPS = 164
NUM_ROLES = 6
CHAMP_DIM = 32
PAIRS = NUM_CHAMPS * NUM_ROLES
OUT = 10
NC, NS = 2, 16
NW = NC * NS

ROWS = 8
COLS = 512


def _pair_table_body(ctT, rtT, w1aT, w1bT, b1c, w2T, b2c, out):
    cpT = jnp.dot(w1aT[...], ctT[...], preferred_element_type=jnp.float32)
    rpT = jnp.dot(w1bT[...], rtT[...], preferred_element_type=jnp.float32)
    pc = lax.broadcasted_iota(jnp.int32, (NUM_CHAMPS, PAIRS), 1) // NUM_ROLES
    ec = (pc == lax.broadcasted_iota(jnp.int32, (NUM_CHAMPS, PAIRS), 0)).astype(jnp.float32)
    pr = lax.broadcasted_iota(jnp.int32, (8, PAIRS), 1) % NUM_ROLES
    er = (pr == lax.broadcasted_iota(jnp.int32, (8, PAIRS), 0)).astype(jnp.float32)
    hT = jnp.maximum(
        jnp.dot(cpT, ec, preferred_element_type=jnp.float32)
        + jnp.dot(rpT, er, preferred_element_type=jnp.float32)
        + b1c[...],
        0.0,
    )
    out[...] = jnp.dot(w2T[...], hT, preferred_element_type=jnp.float32) + b2c[...]


def _build_pair_table(champ_table, role_table, W1, b1, W2, b2):
    ctT = jnp.swapaxes(champ_table, 0, 1)
    rtT = jnp.pad(jnp.swapaxes(role_table, 0, 1), ((0, 0), (0, 8 - NUM_ROLES)))
    W1T = jnp.swapaxes(W1, 0, 1)
    w2T = jnp.swapaxes(W2, 0, 1)
    return pl.pallas_call(
        _pair_table_body,
        out_shape=jax.ShapeDtypeStruct((OUT, PAIRS), jnp.float32),
    )(ctT, rtT, W1T[:, :CHAMP_DIM], W1T[:, CHAMP_DIM:], b1.reshape(-1, 1),
      w2T, b2.reshape(-1, 1))


def _gather_body(chunks_per_w, col_blocks, table_hbm, champ_hbm, role_hbm,
                 out_hbm, table_v, cbuf, rbuf, obuf, csem, rsem, osem):
    wid = lax.axis_index("s") * NC + lax.axis_index("c")
    pltpu.sync_copy(table_hbm, table_v)

    def splat(x):
        return jnp.full((16,), x, jnp.int32)

    def offsets(i):
        t = wid * chunks_per_w + i
        lb = t // col_blocks
        bb = t - lb * col_blocks
        return lb * ROWS, bb * COLS

    def in_copies(i, slot):
        l0, b0 = offsets(i)
        src = lambda ref: ref.at[pl.ds(l0, ROWS), pl.ds(b0, COLS)]
        return (
            pltpu.make_async_copy(src(champ_hbm), cbuf.at[slot], csem.at[slot]),
            pltpu.make_async_copy(src(role_hbm), rbuf.at[slot], rsem.at[slot]),
        )

    def out_copy(i, slot):
        l0, b0 = offsets(i)
        return pltpu.make_async_copy(
            obuf.at[slot], out_hbm.at[:, pl.ds(l0, ROWS), pl.ds(b0, COLS)],
            osem.at[slot])

    for cp in in_copies(0, 0):
        cp.start()

    def chunk_body(i, _):
        slot = lax.rem(i, 2)
        nxt = 1 - slot

        @pl.when(i + 1 < chunks_per_w)
        def _():
            for cp in in_copies(i + 1, nxt):
                cp.start()

        for cp in in_copies(i, slot):
            cp.wait()

        @pl.when(i >= 2)
        def _():
            out_copy(i - 2, slot).wait()

        def row_body(r, _):
            @plsc.parallel_loop(0, COLS // 16, unroll=8)
            def grp_body(g):
                c = cbuf[slot, r, pl.ds(g * 16, 16)]
                rr = rbuf[slot, r, pl.ds(g * 16, 16)]
                idx = c * splat(NUM_ROLES) + rr
                for d in range(OUT):
                    v = plsc.load_gather(table_v, [idx + splat(d * PAIRS)])
                    obuf[slot, d, r, pl.ds(g * 16, 16)] = v

            return 0

        lax.fori_loop(0, ROWS, row_body, 0)
        out_copy(i, slot).start()
        return 0

    lax.fori_loop(0, chunks_per_w, chunk_body, 0)
    out_copy(chunks_per_w - 2, lax.rem(chunks_per_w - 2, 2)).wait()
    out_copy(chunks_per_w - 1, lax.rem(chunks_per_w - 1, 2)).wait()


@functools.cache
def _make_gather(batch, seq_len):
    n_chunks = (seq_len // ROWS) * (batch // COLS)
    assert n_chunks % NW == 0
    chunks_per_w = n_chunks // NW
    col_blocks = batch // COLS
    mesh = plsc.VectorSubcoreMesh(core_axis_name="c", subcore_axis_name="s")
    return pl.kernel(
        functools.partial(_gather_body, chunks_per_w, col_blocks),
        out_type=jax.ShapeDtypeStruct((OUT, seq_len, batch), jnp.float32),
        mesh=mesh,
        compiler_params=pltpu.CompilerParams(needs_layout_passes=False),
        scratch_types=[
            pltpu.VMEM((PAIRS * OUT,), jnp.float32),
            pltpu.VMEM((2, ROWS, COLS), jnp.int32),
            pltpu.VMEM((2, ROWS, COLS), jnp.int32),
            pltpu.VMEM((2, OUT, ROWS, COLS), jnp.float32),
            pltpu.SemaphoreType.DMA((2,)),
            pltpu.SemaphoreType.DMA((2,)),
            pltpu.SemaphoreType.DMA((2,)),
        ],
    )


def kernel(champions, roles, champ_table, role_table, W1, b1, W2, b2):
    B, L = champions.shape
    pair_table = _build_pair_table(champ_table, role_table, W1, b1, W2, b2)
    gather = _make_gather(B, L)
    out_t = gather(pair_table.reshape(-1), jnp.swapaxes(champions, 0, 1),
                   jnp.swapaxes(roles, 0, 1))
    return jnp.transpose(out_t, (2, 1, 0))
